# Initial kernel scaffold; baseline (speedup 1.0000x reference)
import jax
import jax.numpy as jnp
from jax.experimental import pallas as pl


def kernel(x, adj, W, b):
    raise NotImplementedError("write your pallas kernel here")


# speedup vs baseline: 1.0386x; 1.0386x over previous
import functools

import jax
import jax.numpy as jnp
from jax.experimental import pallas as pl


def _sage_block(adj_ref, xb_ref, xf_ref, w1t_ref, w2t_ref, b_ref, out_ref):
    h_n = jnp.dot(adj_ref[...], xf_ref[...], preferred_element_type=jnp.float32)
    acc = jnp.dot(xb_ref[...], w1t_ref[...], preferred_element_type=jnp.float32)
    acc = acc + jnp.dot(h_n, w2t_ref[...], preferred_element_type=jnp.float32)
    out_ref[...] = jnp.maximum(acc + b_ref[...], 0.0)


def _row_block(n: int) -> int:
    for cand in (512, 400, 256, 200, 128, 80, 40, 16, 8):
        if n % cand == 0:
            return cand
    return n


@functools.partial(jax.jit, static_argnames=())
def kernel(x, adj, W, b):
    n, d_in = x.shape
    d_out = W.shape[0]
    w_t = W.T.astype(jnp.float32)
    w1t = w_t[:d_in]
    w2t = w_t[d_in:]
    b2 = b.reshape(1, d_out).astype(jnp.float32)

    bi = _row_block(n)
    grid = (n // bi,)

    return pl.pallas_call(
        _sage_block,
        grid=grid,
        in_specs=[
            pl.BlockSpec((bi, n), lambda i: (i, 0)),
            pl.BlockSpec((bi, d_in), lambda i: (i, 0)),
            pl.BlockSpec((n, d_in), lambda i: (0, 0)),
            pl.BlockSpec((d_in, d_out), lambda i: (0, 0)),
            pl.BlockSpec((d_in, d_out), lambda i: (0, 0)),
            pl.BlockSpec((1, d_out), lambda i: (0, 0)),
        ],
        out_specs=pl.BlockSpec((bi, d_out), lambda i: (i, 0)),
        out_shape=jax.ShapeDtypeStruct((n, d_out), jnp.float32),
    )(adj, x, x, w1t, w2t, b2)
